# Initial kernel scaffold; baseline (speedup 1.0000x reference)
import jax
import jax.numpy as jnp
from jax.experimental import pallas as pl


def kernel(hidden_states, router_w, router_b, W1, b1, W2, b2, ln_g, ln_b):
    raise NotImplementedError("write your pallas kernel here")


# speedup vs baseline: 2.9201x; 2.9201x over previous
import jax
import jax.numpy as jnp
from jax.experimental import pallas as pl
from jax.experimental.pallas import tpu as pltpu

_B, _S, _H = 1, 2048, 768
_F = 3072
_E = 8
_EPS = 1e-12
_TN = 256


def _moe_dense_body(x_ref, rw_ref, rb_ref, w1_ref, b1_ref, w2_ref, b2_ref,
                    g_ref, bb_ref, out_ref, acc_ref):
    e = pl.program_id(0)
    t = pl.program_id(1)
    x = x_ref[...]
    logits = jax.lax.dot_general(
        x.astype(jnp.bfloat16), rw_ref[...].astype(jnp.bfloat16),
        (((1,), (1,)), ((), ())),
        preferred_element_type=jnp.float32) + rb_ref[...]
    iota_e = jax.lax.broadcasted_iota(jnp.int32, logits.shape, 1)
    m0 = jnp.max(logits, axis=-1, keepdims=True)
    e0 = jnp.min(jnp.where(logits >= m0, iota_e, _E), axis=-1, keepdims=True)
    masked = jnp.where(iota_e == e0, -jnp.inf, logits)
    m1 = jnp.max(masked, axis=-1, keepdims=True)
    e1 = jnp.min(jnp.where(masked >= m1, iota_e, _E), axis=-1, keepdims=True)
    w0 = 1.0 / (1.0 + jnp.exp(m1 - m0))
    w1 = 1.0 - w0
    we = jnp.where(e0 == e, w0, jnp.where(e1 == e, w1, 0.0))
    xb = x.astype(jnp.bfloat16)
    h1 = jax.lax.dot_general(
        xb, w1_ref[0], (((1,), (1,)), ((), ())),
        preferred_element_type=jnp.float32) + b1_ref[0]
    h1 = 0.5 * h1 * (1.0 + jax.lax.erf(h1 * 0.7071067811865476))
    y = jax.lax.dot_general(
        h1.astype(jnp.bfloat16), w2_ref[0], (((1,), (1,)), ((), ())),
        preferred_element_type=jnp.float32) + b2_ref[0]
    contrib = we * y

    sl = pl.ds(t * _TN, _TN)

    @pl.when(e == 0)
    def _():
        acc_ref[sl, :] = x + contrib

    @pl.when(e > 0)
    def _():
        acc_ref[sl, :] += contrib

    @pl.when(e == _E - 1)
    def _():
        u = acc_ref[sl, :]
        mu = jnp.mean(u, axis=-1, keepdims=True)
        var = jnp.mean((u - mu) ** 2, axis=-1, keepdims=True)
        out_ref[...] = (u - mu) * jax.lax.rsqrt(var + _EPS) * g_ref[...] + bb_ref[...]


def kernel(hidden_states, router_w, router_b, W1, b1, W2, b2, ln_g, ln_b):
    flat = hidden_states.reshape(_S, _H)
    W1b = W1.astype(jnp.bfloat16)
    W2b = W2.astype(jnp.bfloat16)
    out = pl.pallas_call(
        _moe_dense_body,
        grid=(_E, _S // _TN),
        in_specs=[
            pl.BlockSpec((_TN, _H), lambda e, t: (t, 0)),
            pl.BlockSpec((_E, _H), lambda e, t: (0, 0)),
            pl.BlockSpec((1, _E), lambda e, t: (0, 0)),
            pl.BlockSpec((1, _F, _H), lambda e, t: (e, 0, 0)),
            pl.BlockSpec((1, 1, _F), lambda e, t: (e, 0, 0)),
            pl.BlockSpec((1, _H, _F), lambda e, t: (e, 0, 0)),
            pl.BlockSpec((1, 1, _H), lambda e, t: (e, 0, 0)),
            pl.BlockSpec((1, _H), lambda e, t: (0, 0)),
            pl.BlockSpec((1, _H), lambda e, t: (0, 0)),
        ],
        out_specs=pl.BlockSpec((_TN, _H), lambda e, t: (t, 0)),
        out_shape=jax.ShapeDtypeStruct((_S, _H), jnp.float32),
        scratch_shapes=[pltpu.VMEM((_S, _H), jnp.float32)],
    )(flat, router_w, router_b.reshape(1, _E), W1b, b1.reshape(_E, 1, _F),
      W2b, b2.reshape(_E, 1, _H), ln_g.reshape(1, _H), ln_b.reshape(1, _H))
    return out.reshape(_B, _S, _H)


# speedup vs baseline: 4.0658x; 1.3924x over previous
import jax
import jax.numpy as jnp
from jax import lax
from jax.experimental import pallas as pl
from jax.experimental.pallas import tpu as pltpu
from jax.experimental.pallas import tpu_sc as plsc

_B, _S, _H = 1, 2048, 768
_F = 3072
_E = 8
_EPS = 1e-12
_T = 256
_G = _S * 2 // _T + _E
_P = _G * _T
_NC, _NS = 2, 16
_NW = _NC * _NS
_TPW = _S // _NW
_SQRT1_2 = 0.7071067811865476


def _router_body(x_ref, rw_ref, rb_ref,
                 pos0_ref, pos1_ref, w0_ref, w1_ref, te_ref):
    x = x_ref[...]
    logits = lax.dot_general(
        x.astype(jnp.bfloat16), rw_ref[...].astype(jnp.bfloat16),
        (((1,), (1,)), ((), ())),
        preferred_element_type=jnp.float32) + rb_ref[...]
    iota_e = lax.broadcasted_iota(jnp.int32, logits.shape, 1)
    m0 = jnp.max(logits, axis=-1, keepdims=True)
    e0 = jnp.min(jnp.where(logits >= m0, iota_e, _E), axis=-1, keepdims=True)
    masked = jnp.where(iota_e == e0, -jnp.inf, logits)
    m1 = jnp.max(masked, axis=-1, keepdims=True)
    e1 = jnp.min(jnp.where(masked >= m1, iota_e, _E), axis=-1, keepdims=True)
    w0 = 1.0 / (1.0 + jnp.exp(m1 - m0))
    w0_ref[...] = w0
    w1_ref[...] = 1.0 - w0
    sel = ((iota_e == e0) | (iota_e == e1)).astype(jnp.bfloat16)
    tri = (lax.broadcasted_iota(jnp.int32, (_S, _S), 1)
           < lax.broadcasted_iota(jnp.int32, (_S, _S), 0)).astype(jnp.bfloat16)
    rank = lax.dot_general(tri, sel, (((1,), (0,)), ((), ())),
                           preferred_element_type=jnp.float32)
    count = jnp.sum(sel.astype(jnp.float32), axis=0, keepdims=True)
    pc = ((count.astype(jnp.int32) + _T - 1) // _T) * _T
    trie = (lax.broadcasted_iota(jnp.int32, (_E, _E), 0)
            < lax.broadcasted_iota(jnp.int32, (_E, _E), 1)).astype(jnp.float32)
    pstart = lax.dot_general(pc.astype(jnp.float32), trie,
                             (((1,), (0,)), ((), ())),
                             preferred_element_type=jnp.float32,
                             precision=lax.Precision.HIGHEST)
    rank0 = jnp.sum(jnp.where(iota_e == e0, rank, 0.0), axis=1, keepdims=True)
    rank1 = jnp.sum(jnp.where(iota_e == e1, rank, 0.0), axis=1, keepdims=True)
    ps0 = jnp.sum(jnp.where(iota_e == e0, pstart, 0.0), axis=1, keepdims=True)
    ps1 = jnp.sum(jnp.where(iota_e == e1, pstart, 0.0), axis=1, keepdims=True)
    pos0_ref[...] = (ps0 + rank0).astype(jnp.int32)
    pos1_ref[...] = (ps1 + rank1).astype(jnp.int32)
    psi = pstart.astype(jnp.int32)
    gT = lax.broadcasted_iota(jnp.int32, (_G, _E), 0) * _T
    te = jnp.sum((jnp.broadcast_to(psi, (_G, _E)) <= gT).astype(jnp.int32),
                 axis=1, keepdims=True) - 1
    te = jnp.clip(te, 0, _E - 1)
    n_used = jnp.sum(pc, axis=1, keepdims=True) // _T
    te_ref[...] = jnp.concatenate(
        [te, jnp.broadcast_to(n_used, (8, 1))], axis=0)


def _router(flat, router_w, router_b):
    return pl.pallas_call(
        _router_body,
        grid=(1,),
        in_specs=[
            pl.BlockSpec((_S, _H), lambda i: (0, 0)),
            pl.BlockSpec((_E, _H), lambda i: (0, 0)),
            pl.BlockSpec((1, _E), lambda i: (0, 0)),
        ],
        out_specs=[
            pl.BlockSpec((_S, 1), lambda i: (0, 0)),
            pl.BlockSpec((_S, 1), lambda i: (0, 0)),
            pl.BlockSpec((_S, 1), lambda i: (0, 0)),
            pl.BlockSpec((_S, 1), lambda i: (0, 0)),
            pl.BlockSpec((_G + 8, 1), lambda i: (0, 0)),
        ],
        out_shape=[
            jax.ShapeDtypeStruct((_S, 1), jnp.int32),
            jax.ShapeDtypeStruct((_S, 1), jnp.int32),
            jax.ShapeDtypeStruct((_S, 1), jnp.float32),
            jax.ShapeDtypeStruct((_S, 1), jnp.float32),
            jax.ShapeDtypeStruct((_G + 8, 1), jnp.int32),
        ],
    )(flat, router_w, router_b.reshape(1, _E))


def _sc_scatter_body(flat_hbm, pos0_hbm, pos1_hbm, xs_hbm,
                     rows_v, i0_v, i1_v, sem):
    wid = lax.axis_index("s") * _NC + lax.axis_index("c")
    base = wid * _TPW
    pltpu.sync_copy(flat_hbm.at[pl.ds(base, _TPW)], rows_v)
    pltpu.sync_copy(pos0_hbm.at[pl.ds(base, _TPW)], i0_v)
    pltpu.sync_copy(pos1_hbm.at[pl.ds(base, _TPW)], i1_v)
    pltpu.async_copy(rows_v, xs_hbm.at[i0_v], sem).wait()
    pltpu.async_copy(rows_v, xs_hbm.at[i1_v], sem).wait()


def _sc_scatter(flat, pos0, pos1):
    mesh = plsc.VectorSubcoreMesh(core_axis_name="c", subcore_axis_name="s")
    return pl.kernel(
        _sc_scatter_body,
        out_type=jax.ShapeDtypeStruct((_P, _H), jnp.float32),
        mesh=mesh,
        scratch_types=[
            pltpu.VMEM((_TPW, _H), jnp.float32),
            pltpu.VMEM((_TPW,), jnp.int32),
            pltpu.VMEM((_TPW,), jnp.int32),
            pltpu.SemaphoreType.DMA,
        ],
    )(flat, pos0, pos1)


def _ffn_body(te_ref, xs_ref, w1_ref, b1_ref, w2_ref, b2_ref, y_ref):
    g = pl.program_id(0)

    @pl.when(g < te_ref[_G])
    def _():
        xb = xs_ref[...].astype(jnp.bfloat16)
        h1 = lax.dot_general(xb, w1_ref[0], (((1,), (1,)), ((), ())),
                             preferred_element_type=jnp.float32) + b1_ref[0]
        h1 = 0.5 * h1 * (1.0 + lax.erf(h1 * _SQRT1_2))
        y_ref[...] = lax.dot_general(
            h1.astype(jnp.bfloat16), w2_ref[0], (((1,), (1,)), ((), ())),
            preferred_element_type=jnp.float32) + b2_ref[0]


def _ffn(te_flat, xs, W1b, b1r, W2b, b2r):
    grid_spec = pltpu.PrefetchScalarGridSpec(
        num_scalar_prefetch=1,
        grid=(_G,),
        in_specs=[
            pl.BlockSpec((_T, _H), lambda g, te: (g, 0)),
            pl.BlockSpec((1, _F, _H), lambda g, te: (te[g], 0, 0)),
            pl.BlockSpec((1, 1, _F), lambda g, te: (te[g], 0, 0)),
            pl.BlockSpec((1, _H, _F), lambda g, te: (te[g], 0, 0)),
            pl.BlockSpec((1, 1, _H), lambda g, te: (te[g], 0, 0)),
        ],
        out_specs=pl.BlockSpec((_T, _H), lambda g, te: (g, 0)),
    )
    return pl.pallas_call(
        _ffn_body,
        grid_spec=grid_spec,
        out_shape=jax.ShapeDtypeStruct((_P, _H), jnp.float32),
    )(te_flat, xs, W1b, b1r, W2b, b2r)


def _sc_gather_body(y_hbm, pos0_hbm, pos1_hbm, y0_hbm, y1_hbm,
                    rows_v, i_v, sem):
    wid = lax.axis_index("s") * _NC + lax.axis_index("c")
    base = wid * _TPW
    pltpu.sync_copy(pos0_hbm.at[pl.ds(base, _TPW)], i_v)
    pltpu.async_copy(y_hbm.at[i_v], rows_v, sem).wait()
    pltpu.sync_copy(rows_v, y0_hbm.at[pl.ds(base, _TPW)])
    pltpu.sync_copy(pos1_hbm.at[pl.ds(base, _TPW)], i_v)
    pltpu.async_copy(y_hbm.at[i_v], rows_v, sem).wait()
    pltpu.sync_copy(rows_v, y1_hbm.at[pl.ds(base, _TPW)])


def _sc_gather(y, pos0, pos1):
    mesh = plsc.VectorSubcoreMesh(core_axis_name="c", subcore_axis_name="s")
    return pl.kernel(
        _sc_gather_body,
        out_type=[
            jax.ShapeDtypeStruct((_S, _H), jnp.float32),
            jax.ShapeDtypeStruct((_S, _H), jnp.float32),
        ],
        mesh=mesh,
        scratch_types=[
            pltpu.VMEM((_TPW, _H), jnp.float32),
            pltpu.VMEM((_TPW,), jnp.int32),
            pltpu.SemaphoreType.DMA,
        ],
    )(y, pos0, pos1)


def _combine_body(x_ref, y0_ref, y1_ref, w0_ref, w1_ref, g_ref, b_ref,
                  out_ref):
    u = (x_ref[...] + w0_ref[...] * y0_ref[...] + w1_ref[...] * y1_ref[...])
    mu = jnp.mean(u, axis=-1, keepdims=True)
    var = jnp.mean((u - mu) ** 2, axis=-1, keepdims=True)
    out_ref[...] = (u - mu) * lax.rsqrt(var + _EPS) * g_ref[...] + b_ref[...]


def _combine(flat, y0, y1, w0, w1, ln_g, ln_b):
    tn = 256
    return pl.pallas_call(
        _combine_body,
        grid=(_S // tn,),
        in_specs=[
            pl.BlockSpec((tn, _H), lambda t: (t, 0)),
            pl.BlockSpec((tn, _H), lambda t: (t, 0)),
            pl.BlockSpec((tn, _H), lambda t: (t, 0)),
            pl.BlockSpec((tn, 1), lambda t: (t, 0)),
            pl.BlockSpec((tn, 1), lambda t: (t, 0)),
            pl.BlockSpec((1, _H), lambda t: (0, 0)),
            pl.BlockSpec((1, _H), lambda t: (0, 0)),
        ],
        out_specs=pl.BlockSpec((tn, _H), lambda t: (t, 0)),
        out_shape=jax.ShapeDtypeStruct((_S, _H), jnp.float32),
    )(flat, y0, y1, w0, w1, ln_g.reshape(1, _H), ln_b.reshape(1, _H))


def kernel(hidden_states, router_w, router_b, W1, b1, W2, b2, ln_g, ln_b):
    flat = hidden_states.reshape(_S, _H)
    W1b = W1.astype(jnp.bfloat16)
    W2b = W2.astype(jnp.bfloat16)
    pos0_2d, pos1_2d, w0, w1, te = _router(flat, router_w, router_b)
    pos0 = pos0_2d.reshape(_S)
    pos1 = pos1_2d.reshape(_S)
    xs = _sc_scatter(flat, pos0, pos1)
    y = _ffn(te.reshape(_G + 8), xs, W1b, b1.reshape(_E, 1, _F),
             W2b, b2.reshape(_E, 1, _H))
    y0, y1 = _sc_gather(y, pos0, pos1)
    out = _combine(flat, y0, y1, w0, w1, ln_g, ln_b)
    return out.reshape(_B, _S, _H)


# speedup vs baseline: 5.1453x; 1.2655x over previous
import jax
import jax.numpy as jnp
from jax import lax
from jax.experimental import pallas as pl
from jax.experimental.pallas import tpu as pltpu
from jax.experimental.pallas import tpu_sc as plsc

_B, _S, _H = 1, 2048, 768
_F = 3072
_E = 8
_EPS = 1e-12
_T = 256
_G = _S * 2 // _T + _E
_P = _G * _T
_NC, _NS = 2, 16
_NW = _NC * _NS
_TPW = _S // _NW
_SQRT1_2 = 0.7071067811865476


def _router_body(x_ref, rw_ref, rb_ref,
                 pos0_ref, pos1_ref, w0_ref, w1_ref, te_ref):
    x = x_ref[...]
    logits = lax.dot_general(
        x.astype(jnp.bfloat16), rw_ref[...].astype(jnp.bfloat16),
        (((1,), (1,)), ((), ())),
        preferred_element_type=jnp.float32) + rb_ref[...]
    iota_e = lax.broadcasted_iota(jnp.int32, logits.shape, 1)
    m0 = jnp.max(logits, axis=-1, keepdims=True)
    e0 = jnp.min(jnp.where(logits >= m0, iota_e, _E), axis=-1, keepdims=True)
    masked = jnp.where(iota_e == e0, -jnp.inf, logits)
    m1 = jnp.max(masked, axis=-1, keepdims=True)
    e1 = jnp.min(jnp.where(masked >= m1, iota_e, _E), axis=-1, keepdims=True)
    w0 = 1.0 / (1.0 + jnp.exp(m1 - m0))
    w0_ref[...] = w0
    w1_ref[...] = 1.0 - w0
    sel = ((iota_e == e0) | (iota_e == e1)).astype(jnp.bfloat16)
    tri = (lax.broadcasted_iota(jnp.int32, (_S, _S), 1)
           < lax.broadcasted_iota(jnp.int32, (_S, _S), 0)).astype(jnp.bfloat16)
    rank = lax.dot_general(tri, sel, (((1,), (0,)), ((), ())),
                           preferred_element_type=jnp.float32)
    count = jnp.sum(sel.astype(jnp.float32), axis=0, keepdims=True)
    pc = ((count.astype(jnp.int32) + _T - 1) // _T) * _T
    trie = (lax.broadcasted_iota(jnp.int32, (_E, _E), 0)
            < lax.broadcasted_iota(jnp.int32, (_E, _E), 1)).astype(jnp.float32)
    pstart = lax.dot_general(pc.astype(jnp.float32), trie,
                             (((1,), (0,)), ((), ())),
                             preferred_element_type=jnp.float32,
                             precision=lax.Precision.HIGHEST)
    rank0 = jnp.sum(jnp.where(iota_e == e0, rank, 0.0), axis=1, keepdims=True)
    rank1 = jnp.sum(jnp.where(iota_e == e1, rank, 0.0), axis=1, keepdims=True)
    ps0 = jnp.sum(jnp.where(iota_e == e0, pstart, 0.0), axis=1, keepdims=True)
    ps1 = jnp.sum(jnp.where(iota_e == e1, pstart, 0.0), axis=1, keepdims=True)
    pos0_ref[...] = (ps0 + rank0).astype(jnp.int32)
    pos1_ref[...] = (ps1 + rank1).astype(jnp.int32)
    psi = pstart.astype(jnp.int32)
    gT = lax.broadcasted_iota(jnp.int32, (_G, _E), 0) * _T
    te = jnp.sum((jnp.broadcast_to(psi, (_G, _E)) <= gT).astype(jnp.int32),
                 axis=1, keepdims=True) - 1
    te = jnp.clip(te, 0, _E - 1)
    n_used = jnp.sum(pc, axis=1, keepdims=True) // _T
    te_ref[...] = jnp.concatenate(
        [te, jnp.broadcast_to(n_used, (8, 1))], axis=0)


def _router(flat, router_w, router_b):
    return pl.pallas_call(
        _router_body,
        grid=(1,),
        in_specs=[
            pl.BlockSpec((_S, _H), lambda i: (0, 0)),
            pl.BlockSpec((_E, _H), lambda i: (0, 0)),
            pl.BlockSpec((1, _E), lambda i: (0, 0)),
        ],
        out_specs=[
            pl.BlockSpec((_S, 1), lambda i: (0, 0)),
            pl.BlockSpec((_S, 1), lambda i: (0, 0)),
            pl.BlockSpec((_S, 1), lambda i: (0, 0)),
            pl.BlockSpec((_S, 1), lambda i: (0, 0)),
            pl.BlockSpec((_G + 8, 1), lambda i: (0, 0)),
        ],
        out_shape=[
            jax.ShapeDtypeStruct((_S, 1), jnp.int32),
            jax.ShapeDtypeStruct((_S, 1), jnp.int32),
            jax.ShapeDtypeStruct((_S, 1), jnp.float32),
            jax.ShapeDtypeStruct((_S, 1), jnp.float32),
            jax.ShapeDtypeStruct((_G + 8, 1), jnp.int32),
        ],
    )(flat, router_w, router_b.reshape(1, _E))


def _sc_scatter_body(flat_hbm, pos0_hbm, pos1_hbm, xs_hbm,
                     rows_v, i0_v, i1_v, sem):
    wid = lax.axis_index("s") * _NC + lax.axis_index("c")
    base = wid * _TPW
    pltpu.sync_copy(flat_hbm.at[pl.ds(base, _TPW)], rows_v)
    pltpu.sync_copy(pos0_hbm.at[pl.ds(base, _TPW)], i0_v)
    pltpu.sync_copy(pos1_hbm.at[pl.ds(base, _TPW)], i1_v)
    pltpu.async_copy(rows_v, xs_hbm.at[i0_v], sem).wait()
    pltpu.async_copy(rows_v, xs_hbm.at[i1_v], sem).wait()


def _sc_scatter(flat, pos0, pos1):
    mesh = plsc.VectorSubcoreMesh(core_axis_name="c", subcore_axis_name="s")
    return pl.kernel(
        _sc_scatter_body,
        out_type=jax.ShapeDtypeStruct((_P, _H), jnp.float32),
        mesh=mesh,
        scratch_types=[
            pltpu.VMEM((_TPW, _H), jnp.float32),
            pltpu.VMEM((_TPW,), jnp.int32),
            pltpu.VMEM((_TPW,), jnp.int32),
            pltpu.SemaphoreType.DMA,
        ],
    )(flat, pos0, pos1)


def _ffn_body(te_ref, xs_ref, w1_ref, b1_ref, w2_ref, b2_ref, y_ref):
    g = pl.program_id(0)

    @pl.when(g < te_ref[_G])
    def _():
        xb = xs_ref[...].astype(jnp.bfloat16)
        h1 = lax.dot_general(xb, w1_ref[0].astype(jnp.bfloat16),
                             (((1,), (1,)), ((), ())),
                             preferred_element_type=jnp.float32) + b1_ref[0]
        h1 = 0.5 * h1 * (1.0 + lax.erf(h1 * _SQRT1_2))
        y_ref[...] = lax.dot_general(
            h1.astype(jnp.bfloat16), w2_ref[0].astype(jnp.bfloat16),
            (((1,), (1,)), ((), ())),
            preferred_element_type=jnp.float32) + b2_ref[0]


def _ffn(te_flat, xs, W1b, b1r, W2b, b2r):
    grid_spec = pltpu.PrefetchScalarGridSpec(
        num_scalar_prefetch=1,
        grid=(_G,),
        in_specs=[
            pl.BlockSpec((_T, _H), lambda g, te: (g, 0)),
            pl.BlockSpec((1, _F, _H), lambda g, te: (te[g], 0, 0)),
            pl.BlockSpec((1, 1, _F), lambda g, te: (te[g], 0, 0)),
            pl.BlockSpec((1, _H, _F), lambda g, te: (te[g], 0, 0)),
            pl.BlockSpec((1, 1, _H), lambda g, te: (te[g], 0, 0)),
        ],
        out_specs=pl.BlockSpec((_T, _H), lambda g, te: (g, 0)),
    )
    return pl.pallas_call(
        _ffn_body,
        grid_spec=grid_spec,
        out_shape=jax.ShapeDtypeStruct((_P, _H), jnp.float32),
    )(te_flat, xs, W1b, b1r, W2b, b2r)


def _sc_gather_body(y_hbm, pos0_hbm, pos1_hbm, y0_hbm, y1_hbm,
                    rows_v, i_v, sem):
    wid = lax.axis_index("s") * _NC + lax.axis_index("c")
    base = wid * _TPW
    pltpu.sync_copy(pos0_hbm.at[pl.ds(base, _TPW)], i_v)
    pltpu.async_copy(y_hbm.at[i_v], rows_v, sem).wait()
    pltpu.sync_copy(rows_v, y0_hbm.at[pl.ds(base, _TPW)])
    pltpu.sync_copy(pos1_hbm.at[pl.ds(base, _TPW)], i_v)
    pltpu.async_copy(y_hbm.at[i_v], rows_v, sem).wait()
    pltpu.sync_copy(rows_v, y1_hbm.at[pl.ds(base, _TPW)])


def _sc_gather(y, pos0, pos1):
    mesh = plsc.VectorSubcoreMesh(core_axis_name="c", subcore_axis_name="s")
    return pl.kernel(
        _sc_gather_body,
        out_type=[
            jax.ShapeDtypeStruct((_S, _H), jnp.float32),
            jax.ShapeDtypeStruct((_S, _H), jnp.float32),
        ],
        mesh=mesh,
        scratch_types=[
            pltpu.VMEM((_TPW, _H), jnp.float32),
            pltpu.VMEM((_TPW,), jnp.int32),
            pltpu.SemaphoreType.DMA,
        ],
    )(y, pos0, pos1)


def _combine_body(x_ref, y0_ref, y1_ref, w0_ref, w1_ref, g_ref, b_ref,
                  out_ref):
    u = (x_ref[...] + w0_ref[...] * y0_ref[...] + w1_ref[...] * y1_ref[...])
    mu = jnp.mean(u, axis=-1, keepdims=True)
    var = jnp.mean((u - mu) ** 2, axis=-1, keepdims=True)
    out_ref[...] = (u - mu) * lax.rsqrt(var + _EPS) * g_ref[...] + b_ref[...]


def _combine(flat, y0, y1, w0, w1, ln_g, ln_b):
    tn = 256
    return pl.pallas_call(
        _combine_body,
        grid=(_S // tn,),
        in_specs=[
            pl.BlockSpec((tn, _H), lambda t: (t, 0)),
            pl.BlockSpec((tn, _H), lambda t: (t, 0)),
            pl.BlockSpec((tn, _H), lambda t: (t, 0)),
            pl.BlockSpec((tn, 1), lambda t: (t, 0)),
            pl.BlockSpec((tn, 1), lambda t: (t, 0)),
            pl.BlockSpec((1, _H), lambda t: (0, 0)),
            pl.BlockSpec((1, _H), lambda t: (0, 0)),
        ],
        out_specs=pl.BlockSpec((tn, _H), lambda t: (t, 0)),
        out_shape=jax.ShapeDtypeStruct((_S, _H), jnp.float32),
    )(flat, y0, y1, w0, w1, ln_g.reshape(1, _H), ln_b.reshape(1, _H))


def kernel(hidden_states, router_w, router_b, W1, b1, W2, b2, ln_g, ln_b):
    flat = hidden_states.reshape(_S, _H)
    pos0_2d, pos1_2d, w0, w1, te = _router(flat, router_w, router_b)
    pos0 = pos0_2d.reshape(_S)
    pos1 = pos1_2d.reshape(_S)
    xs = _sc_scatter(flat, pos0, pos1)
    y = _ffn(te.reshape(_G + 8), xs, W1, b1.reshape(_E, 1, _F),
             W2, b2.reshape(_E, 1, _H))
    y0, y1 = _sc_gather(y, pos0, pos1)
    out = _combine(flat, y0, y1, w0, w1, ln_g, ln_b)
    return out.reshape(_B, _S, _H)
